# Initial kernel scaffold; baseline (speedup 1.0000x reference)
import jax
import jax.numpy as jnp
from jax.experimental import pallas as pl


def kernel(x, edge_index, W1_rel, b1, W1_root, W2_rel, b2, W2_root, W3_rel, b3, W3_root):
    raise NotImplementedError("write your pallas kernel here")


# speedup vs baseline: 4.5439x; 4.5439x over previous
import functools

import jax
import jax.numpy as jnp
from jax import lax
from jax.experimental import pallas as pl
from jax.experimental.pallas import tpu as pltpu
from jax.experimental.pallas import tpu_sc as plsc

NC = 2
NS = 16
NW = NC * NS
L = 16
K = 128

_HIGHEST = lax.Precision.HIGHEST


@functools.lru_cache(maxsize=None)
def _make_agg(n_nodes, wp, n_chunks, n_panels):
    assert wp % L == 0 and n_chunks % 2 == 0
    stripe = -(-(n_nodes + 1) // (NS * K)) * K
    n_acc = stripe * NS

    mesh = plsc.VectorSubcoreMesh(core_axis_name="c", subcore_axis_name="s")

    def body(*refs):
        ys = refs[:n_panels]
        (src_hbm, dst_hbm, out_hbm,
         srcv, dstv, rows_a, rows_b, zbuf, acc, sem_a, sem_b) = refs[n_panels:]
        cid = lax.axis_index("c")
        sid = lax.axis_index("s")
        wid = sid * NC + cid

        @pl.loop(0, K)
        def _zero(r):
            for c in range(wp // L):
                zbuf[r, pl.ds(c * L, L)] = jnp.zeros((L,), jnp.float32)

        pltpu.sync_copy(src_hbm.at[wid], srcv)
        pltpu.sync_copy(dst_hbm.at[wid], dstv)

        for p in range(n_panels):
            y_hbm = ys[p]

            for j in range(stripe // K):
                pltpu.sync_copy(zbuf, acc.at[pl.ds(sid * stripe + j * K, K)])
            plsc.subcore_barrier()

            def fire(j, rbuf, sem):
                pltpu.async_copy(y_hbm.at[srcv.at[j]], rbuf, sem)

            def drain(rbuf, sem):
                pltpu.make_async_copy(y_hbm.at[srcv.at[0]], rbuf, sem).wait()

            def scat(j, rbuf):
                pltpu.sync_copy(rbuf, acc.at[dstv.at[j]], add=True)

            fire(0, rows_a, sem_a)

            @pl.loop(0, n_chunks // 2 - 1)
            def _main(t):
                j = 2 * t
                fire(j + 1, rows_b, sem_b)
                drain(rows_a, sem_a)
                scat(j, rows_a)
                fire(j + 2, rows_a, sem_a)
                drain(rows_b, sem_b)
                scat(j + 1, rows_b)

            fire(n_chunks - 1, rows_b, sem_b)
            drain(rows_a, sem_a)
            scat(n_chunks - 2, rows_a)
            drain(rows_b, sem_b)
            scat(n_chunks - 1, rows_b)

            plsc.subcore_barrier()

            r0 = sid * stripe
            pltpu.sync_copy(acc.at[pl.ds(r0, stripe)],
                            out_hbm.at[cid, p, pl.ds(r0, stripe)])

    return pl.kernel(
        body,
        jax.ShapeDtypeStruct((NC, n_panels, n_acc, wp), jnp.float32),
        mesh=mesh,
        compiler_params=pltpu.CompilerParams(use_tc_tiling_on_sc=False),
        scratch_types=[
            pltpu.VMEM((n_chunks, K), jnp.int32),
            pltpu.VMEM((n_chunks, K), jnp.int32),
            pltpu.VMEM((K, wp), jnp.float32),
            pltpu.VMEM((K, wp), jnp.float32),
            pltpu.VMEM((K, wp), jnp.float32),
            pltpu.VMEM_SHARED((n_acc, wp), jnp.float32),
            pltpu.SemaphoreType.DMA,
            pltpu.SemaphoreType.DMA,
        ],
    )


def _dot(a, b):
    return jnp.dot(a, b, preferred_element_type=jnp.float32, precision=_HIGHEST)


def _cat_partials(agg_ref):
    a = agg_ref[0] + agg_ref[1]
    return jnp.concatenate([a[p] for p in range(a.shape[0])], axis=1)


def _tc_layer1(agg, x, w_rel, w_root, b, r):
    n, d = x.shape
    dh = w_rel.shape[1]
    np_, wp = agg.shape[1], agg.shape[3]

    def body(agg_ref, x_ref, wr_ref, wo_ref, b_ref, out_ref):
        a = _cat_partials(agg_ref)
        h = _dot(a, wr_ref[...]) + _dot(x_ref[...], wo_ref[...]) + b_ref[...]
        h = jnp.maximum(h, 0.0)
        out_ref[0] = h[:, :wp]
        out_ref[1] = h[:, wp:]

    return pl.pallas_call(
        body,
        grid=(n // r,),
        in_specs=[
            pl.BlockSpec((2, np_, r, wp), lambda i: (0, 0, i, 0)),
            pl.BlockSpec((r, d), lambda i: (i, 0)),
            pl.BlockSpec((d, dh), lambda i: (0, 0)),
            pl.BlockSpec((d, dh), lambda i: (0, 0)),
            pl.BlockSpec((1, dh), lambda i: (0, 0)),
        ],
        out_specs=pl.BlockSpec((2, r, wp), lambda i: (0, i, 0)),
        out_shape=jax.ShapeDtypeStruct((2, n, wp), jnp.float32),
    )(agg, x, w_rel, w_root, b[None])


def _tc_layer2(agg, h1s, w_rel, w_root, b, w3_rel_pad, r):
    _, n, wp = h1s.shape
    d = 2 * wp
    de = w_rel.shape[1]
    wc = w3_rel_pad.shape[1]
    np_ = agg.shape[1]

    def body(agg_ref, h1_ref, wr_ref, wo_ref, b_ref, w3_ref, h2_ref, y3_ref):
        a = _cat_partials(agg_ref)
        h1 = jnp.concatenate([h1_ref[0], h1_ref[1]], axis=1)
        h = _dot(a, wr_ref[...]) + _dot(h1, wo_ref[...]) + b_ref[...]
        h2 = jnp.maximum(h, 0.0)
        h2_ref[...] = h2
        y3_ref[...] = _dot(h2, w3_ref[...])

    return pl.pallas_call(
        body,
        grid=(n // r,),
        in_specs=[
            pl.BlockSpec((2, np_, r, wp), lambda i: (0, 0, i, 0)),
            pl.BlockSpec((2, r, wp), lambda i: (0, i, 0)),
            pl.BlockSpec((d, de), lambda i: (0, 0)),
            pl.BlockSpec((d, de), lambda i: (0, 0)),
            pl.BlockSpec((1, de), lambda i: (0, 0)),
            pl.BlockSpec((de, wc), lambda i: (0, 0)),
        ],
        out_specs=[
            pl.BlockSpec((r, de), lambda i: (i, 0)),
            pl.BlockSpec((r, wc), lambda i: (i, 0)),
        ],
        out_shape=[
            jax.ShapeDtypeStruct((n, de), jnp.float32),
            jax.ShapeDtypeStruct((n, wc), jnp.float32),
        ],
    )(agg, h1s, w_rel, w_root, b[None], w3_rel_pad)


def _tc_layer3(agg, h2, w_root_pad, b_pad, n_classes, r):
    n, de = h2.shape
    wc = w_root_pad.shape[1]

    def body(agg_ref, h2_ref, wo_ref, b_ref, out_ref):
        a = agg_ref[0, 0] + agg_ref[1, 0]
        logits = a + _dot(h2_ref[...], wo_ref[...]) + b_ref[...]
        mask = lax.broadcasted_iota(jnp.int32, logits.shape, 1) < n_classes
        masked = jnp.where(mask, logits, -jnp.inf)
        m = jnp.max(masked, axis=1, keepdims=True)
        e = jnp.where(mask, jnp.exp(logits - m), 0.0)
        s = jnp.sum(e, axis=1, keepdims=True)
        out_ref[...] = logits - m - jnp.log(s)

    return pl.pallas_call(
        body,
        grid=(n // r,),
        in_specs=[
            pl.BlockSpec((2, 1, r, wc), lambda i: (0, 0, i, 0)),
            pl.BlockSpec((r, de), lambda i: (i, 0)),
            pl.BlockSpec((de, wc), lambda i: (0, 0)),
            pl.BlockSpec((1, wc), lambda i: (0, 0)),
        ],
        out_specs=pl.BlockSpec((r, wc), lambda i: (i, 0)),
        out_shape=jax.ShapeDtypeStruct((n, wc), jnp.float32),
    )(agg, h2, w_root_pad, b_pad[None])


def kernel(x, edge_index, W1_rel, b1, W1_root, W2_rel, b2, W2_root, W3_rel, b3, W3_root):
    n, d_in = x.shape
    e = edge_index.shape[1]
    n_classes = W3_rel.shape[1]
    d_hid = W1_rel.shape[1]
    assert d_hid == d_in and d_in % 2 == 0
    wp = d_in // 2
    r = 1000
    assert n % r == 0

    epw = -(-e // (NW * 2 * K)) * 2 * K
    e_pad = epw * NW
    n_chunks = epw // K
    src = jnp.pad(edge_index[0], (0, e_pad - e)).reshape(NW, n_chunks, K)
    dst = jnp.pad(edge_index[1], (0, e_pad - e),
                  constant_values=n).reshape(NW, n_chunks, K)

    wc = -(-n_classes // L) * L
    W3r = jnp.pad(W3_rel, ((0, 0), (0, wc - n_classes)))
    W3o = jnp.pad(W3_root, ((0, 0), (0, wc - n_classes)))
    b3p = jnp.pad(b3, (0, wc - n_classes))

    agg_wide = _make_agg(n, wp, n_chunks, 2)
    agg_narrow = _make_agg(n, wc, n_chunks, 1)

    agg1 = agg_wide(x[:, :wp], x[:, wp:], src, dst)
    h1s = _tc_layer1(agg1, x, W1_rel, W1_root, b1, r)
    agg2 = agg_wide(h1s[0], h1s[1], src, dst)
    h2, y3 = _tc_layer2(agg2, h1s, W2_rel, W2_root, b2, W3r, r)
    agg3 = agg_narrow(y3, src, dst)
    out = _tc_layer3(agg3, h2, W3o, b3p, n_classes, r)
    return out[:, :n_classes]


# speedup vs baseline: 4.5740x; 1.0066x over previous
import functools

import jax
import jax.numpy as jnp
from jax import lax
from jax.experimental import pallas as pl
from jax.experimental.pallas import tpu as pltpu
from jax.experimental.pallas import tpu_sc as plsc

NC = 2
NS = 16
NW = NC * NS
L = 16
K = 128

_HIGHEST = lax.Precision.HIGHEST


@functools.lru_cache(maxsize=None)
def _make_agg(n_nodes, wp, n_chunks, n_panels):
    assert wp % L == 0 and n_chunks % 4 == 0
    stripe = -(-(n_nodes + 1) // (NS * K)) * K
    n_acc = stripe * NS

    mesh = plsc.VectorSubcoreMesh(core_axis_name="c", subcore_axis_name="s")

    NB = 4

    def body(*refs):
        ys = refs[:n_panels]
        (src_hbm, dst_hbm, out_hbm,
         srcv, dstv, rows, zbuf, acc, gsems, ssems) = refs[n_panels:]
        cid = lax.axis_index("c")
        sid = lax.axis_index("s")
        wid = sid * NC + cid

        @pl.loop(0, K)
        def _zero(r):
            for c in range(wp // L):
                zbuf[r, pl.ds(c * L, L)] = jnp.zeros((L,), jnp.float32)

        pltpu.sync_copy(src_hbm.at[wid], srcv)
        pltpu.sync_copy(dst_hbm.at[wid], dstv)

        for p in range(n_panels):
            y_hbm = ys[p]

            for j in range(stripe // K):
                pltpu.sync_copy(zbuf, acc.at[pl.ds(sid * stripe + j * K, K)])
            plsc.subcore_barrier()

            def fire_g(j, b):
                pltpu.async_copy(y_hbm.at[srcv.at[j]], rows.at[b], gsems.at[b])

            def drain_g(b):
                pltpu.make_async_copy(y_hbm.at[srcv.at[0]], rows.at[b],
                                      gsems.at[b]).wait()

            def fire_s(j, b):
                pltpu.async_copy(rows.at[b], acc.at[dstv.at[j]], ssems.at[b],
                                 add=True)

            def drain_s(b):
                pltpu.make_async_copy(rows.at[b], acc.at[dstv.at[0]],
                                      ssems.at[b]).wait()

            for b in range(NB):
                fire_g(b, b)

            @pl.loop(0, n_chunks // NB - 1)
            def _main(q):
                t = NB * q
                for b in range(NB):
                    drain_g(b)
                    fire_s(t + b, b)
                    drain_s(b)
                    fire_g(t + b + NB, b)

            for b in range(NB):
                drain_g(b)
                fire_s(n_chunks - NB + b, b)
            for b in range(NB):
                drain_s(b)

            plsc.subcore_barrier()

            r0 = sid * stripe
            pltpu.sync_copy(acc.at[pl.ds(r0, stripe)],
                            out_hbm.at[cid, p, pl.ds(r0, stripe)])

    return pl.kernel(
        body,
        jax.ShapeDtypeStruct((NC, n_panels, n_acc, wp), jnp.float32),
        mesh=mesh,
        compiler_params=pltpu.CompilerParams(use_tc_tiling_on_sc=False),
        scratch_types=[
            pltpu.VMEM((n_chunks, K), jnp.int32),
            pltpu.VMEM((n_chunks, K), jnp.int32),
            pltpu.VMEM((4, K, wp), jnp.float32),
            pltpu.VMEM((K, wp), jnp.float32),
            pltpu.VMEM_SHARED((n_acc, wp), jnp.float32),
            pltpu.SemaphoreType.DMA((4,)),
            pltpu.SemaphoreType.DMA((4,)),
        ],
    )


def _dot(a, b):
    return jnp.dot(a, b, preferred_element_type=jnp.float32, precision=_HIGHEST)


def _cat_partials(agg_ref):
    a = agg_ref[0] + agg_ref[1]
    return jnp.concatenate([a[p] for p in range(a.shape[0])], axis=1)


def _tc_layer1(agg, x, w_rel, w_root, b, r):
    n, d = x.shape
    dh = w_rel.shape[1]
    np_, wp = agg.shape[1], agg.shape[3]

    def body(agg_ref, x_ref, wr_ref, wo_ref, b_ref, out_ref):
        a = _cat_partials(agg_ref)
        h = _dot(a, wr_ref[...]) + _dot(x_ref[...], wo_ref[...]) + b_ref[...]
        h = jnp.maximum(h, 0.0)
        out_ref[0] = h[:, :wp]
        out_ref[1] = h[:, wp:]

    return pl.pallas_call(
        body,
        grid=(n // r,),
        in_specs=[
            pl.BlockSpec((2, np_, r, wp), lambda i: (0, 0, i, 0)),
            pl.BlockSpec((r, d), lambda i: (i, 0)),
            pl.BlockSpec((d, dh), lambda i: (0, 0)),
            pl.BlockSpec((d, dh), lambda i: (0, 0)),
            pl.BlockSpec((1, dh), lambda i: (0, 0)),
        ],
        out_specs=pl.BlockSpec((2, r, wp), lambda i: (0, i, 0)),
        out_shape=jax.ShapeDtypeStruct((2, n, wp), jnp.float32),
    )(agg, x, w_rel, w_root, b[None])


def _tc_layer2(agg, h1s, w_rel, w_root, b, w3_rel_pad, r):
    _, n, wp = h1s.shape
    d = 2 * wp
    de = w_rel.shape[1]
    wc = w3_rel_pad.shape[1]
    np_ = agg.shape[1]

    def body(agg_ref, h1_ref, wr_ref, wo_ref, b_ref, w3_ref, h2_ref, y3_ref):
        a = _cat_partials(agg_ref)
        h1 = jnp.concatenate([h1_ref[0], h1_ref[1]], axis=1)
        h = _dot(a, wr_ref[...]) + _dot(h1, wo_ref[...]) + b_ref[...]
        h2 = jnp.maximum(h, 0.0)
        h2_ref[...] = h2
        y3_ref[...] = _dot(h2, w3_ref[...])

    return pl.pallas_call(
        body,
        grid=(n // r,),
        in_specs=[
            pl.BlockSpec((2, np_, r, wp), lambda i: (0, 0, i, 0)),
            pl.BlockSpec((2, r, wp), lambda i: (0, i, 0)),
            pl.BlockSpec((d, de), lambda i: (0, 0)),
            pl.BlockSpec((d, de), lambda i: (0, 0)),
            pl.BlockSpec((1, de), lambda i: (0, 0)),
            pl.BlockSpec((de, wc), lambda i: (0, 0)),
        ],
        out_specs=[
            pl.BlockSpec((r, de), lambda i: (i, 0)),
            pl.BlockSpec((r, wc), lambda i: (i, 0)),
        ],
        out_shape=[
            jax.ShapeDtypeStruct((n, de), jnp.float32),
            jax.ShapeDtypeStruct((n, wc), jnp.float32),
        ],
    )(agg, h1s, w_rel, w_root, b[None], w3_rel_pad)


def _tc_layer3(agg, h2, w_root_pad, b_pad, n_classes, r):
    n, de = h2.shape
    wc = w_root_pad.shape[1]

    def body(agg_ref, h2_ref, wo_ref, b_ref, out_ref):
        a = agg_ref[0, 0] + agg_ref[1, 0]
        logits = a + _dot(h2_ref[...], wo_ref[...]) + b_ref[...]
        mask = lax.broadcasted_iota(jnp.int32, logits.shape, 1) < n_classes
        masked = jnp.where(mask, logits, -jnp.inf)
        m = jnp.max(masked, axis=1, keepdims=True)
        e = jnp.where(mask, jnp.exp(logits - m), 0.0)
        s = jnp.sum(e, axis=1, keepdims=True)
        out_ref[...] = logits - m - jnp.log(s)

    return pl.pallas_call(
        body,
        grid=(n // r,),
        in_specs=[
            pl.BlockSpec((2, 1, r, wc), lambda i: (0, 0, i, 0)),
            pl.BlockSpec((r, de), lambda i: (i, 0)),
            pl.BlockSpec((de, wc), lambda i: (0, 0)),
            pl.BlockSpec((1, wc), lambda i: (0, 0)),
        ],
        out_specs=pl.BlockSpec((r, wc), lambda i: (i, 0)),
        out_shape=jax.ShapeDtypeStruct((n, wc), jnp.float32),
    )(agg, h2, w_root_pad, b_pad[None])


def kernel(x, edge_index, W1_rel, b1, W1_root, W2_rel, b2, W2_root, W3_rel, b3, W3_root):
    n, d_in = x.shape
    e = edge_index.shape[1]
    n_classes = W3_rel.shape[1]
    d_hid = W1_rel.shape[1]
    assert d_hid == d_in and d_in % 2 == 0
    wp = d_in // 2
    r = 1000
    assert n % r == 0

    epw = -(-e // (NW * 2 * K)) * 2 * K
    e_pad = epw * NW
    n_chunks = epw // K
    src = jnp.pad(edge_index[0], (0, e_pad - e)).reshape(NW, n_chunks, K)
    dst = jnp.pad(edge_index[1], (0, e_pad - e),
                  constant_values=n).reshape(NW, n_chunks, K)

    wc = -(-n_classes // L) * L
    W3r = jnp.pad(W3_rel, ((0, 0), (0, wc - n_classes)))
    W3o = jnp.pad(W3_root, ((0, 0), (0, wc - n_classes)))
    b3p = jnp.pad(b3, (0, wc - n_classes))

    agg_wide = _make_agg(n, wp, n_chunks, 2)
    agg_narrow = _make_agg(n, wc, n_chunks, 1)

    agg1 = agg_wide(x[:, :wp], x[:, wp:], src, dst)
    h1s = _tc_layer1(agg1, x, W1_rel, W1_root, b1, r)
    agg2 = agg_wide(h1s[0], h1s[1], src, dst)
    h2, y3 = _tc_layer2(agg2, h1s, W2_rel, W2_root, b2, W3r, r)
    agg3 = agg_narrow(y3, src, dst)
    out = _tc_layer3(agg3, h2, W3o, b3p, n_classes, r)
    return out[:, :n_classes]


# speedup vs baseline: 8.7724x; 1.9179x over previous
import functools

import jax
import jax.numpy as jnp
from jax import lax
from jax.experimental import pallas as pl
from jax.experimental.pallas import tpu as pltpu
from jax.experimental.pallas import tpu_sc as plsc

NC = 2
NS = 16
NW = NC * NS
L = 16
K = 128
WP = 32
NB = 4

_HIGHEST = lax.Precision.HIGHEST


@functools.lru_cache(maxsize=None)
def _make_agg(n_nodes, n_chunks, n_panels):
    assert n_chunks % NB == 0 and n_nodes % NS == 0
    stripe = -(-(n_nodes + 1) // (NS * K)) * K
    n_acc = stripe * NS
    rows_pt = n_nodes // NS

    mesh = plsc.VectorSubcoreMesh(core_axis_name="c", subcore_axis_name="s")

    def body(y_hbm, src_hbm, dst_hbm, out_hbm,
             srcv, dstv, rows, zbuf, acc, ysp, gsems, ssems):
        cid = lax.axis_index("c")
        sid = lax.axis_index("s")
        wid = sid * NC + cid

        @pl.loop(0, K)
        def _zero(r):
            for c in range(WP // L):
                zbuf[r, pl.ds(c * L, L)] = jnp.zeros((L,), jnp.float32)

        pltpu.sync_copy(src_hbm.at[wid], srcv)
        pltpu.sync_copy(dst_hbm.at[wid], dstv)

        def pipeline(src_of):
            def fire_g(j, b):
                pltpu.async_copy(src_of.at[srcv.at[j]], rows.at[b], gsems.at[b])

            def drain_g(b):
                pltpu.make_async_copy(src_of.at[srcv.at[0]], rows.at[b],
                                      gsems.at[b]).wait()

            def fire_s(j, b):
                pltpu.async_copy(rows.at[b], acc.at[dstv.at[j]], ssems.at[b],
                                 add=True)

            def drain_s(b):
                pltpu.make_async_copy(rows.at[b], acc.at[dstv.at[0]],
                                      ssems.at[b]).wait()

            for b in range(NB):
                fire_g(b, b)

            @pl.loop(0, n_chunks // NB - 1)
            def _main(q):
                t = NB * q
                for b in range(NB):
                    drain_g(b)
                    fire_s(t + b, b)
                    drain_s(b)
                    fire_g(t + b + NB, b)

            for b in range(NB):
                drain_g(b)
                fire_s(n_chunks - NB + b, b)
            for b in range(NB):
                drain_s(b)

        @pl.loop(0, n_panels)
        def _panel(p):
            for j in range(stripe // K):
                pltpu.sync_copy(zbuf, acc.at[pl.ds(sid * stripe + j * K, K)])

            @pl.when(cid == 1)
            def _():
                r0 = sid * rows_pt
                pltpu.sync_copy(y_hbm.at[p, pl.ds(r0, rows_pt)],
                                ysp.at[pl.ds(r0, rows_pt)])

            plsc.subcore_barrier()

            @pl.when(cid == 0)
            def _():
                pipeline(y_hbm.at[p])

            @pl.when(cid == 1)
            def _():
                pipeline(ysp)

            plsc.subcore_barrier()

            r0 = sid * stripe
            pltpu.sync_copy(acc.at[pl.ds(r0, stripe)],
                            out_hbm.at[cid, p, pl.ds(r0, stripe)])
            plsc.subcore_barrier()

    return pl.kernel(
        body,
        jax.ShapeDtypeStruct((NC, n_panels, n_acc, WP), jnp.float32),
        mesh=mesh,
        compiler_params=pltpu.CompilerParams(use_tc_tiling_on_sc=False),
        scratch_types=[
            pltpu.VMEM((n_chunks, K), jnp.int32),
            pltpu.VMEM((n_chunks, K), jnp.int32),
            pltpu.VMEM((NB, K, WP), jnp.float32),
            pltpu.VMEM((K, WP), jnp.float32),
            pltpu.VMEM_SHARED((n_acc, WP), jnp.float32),
            pltpu.VMEM_SHARED((n_nodes, WP), jnp.float32),
            pltpu.SemaphoreType.DMA((NB,)),
            pltpu.SemaphoreType.DMA((NB,)),
        ],
    )


def _dot(a, b):
    return jnp.dot(a, b, preferred_element_type=jnp.float32, precision=_HIGHEST)


def _cat_partials(agg_ref):
    a = agg_ref[0] + agg_ref[1]
    return jnp.concatenate([a[p] for p in range(a.shape[0])], axis=1)


def _split_store(out_ref, h, npan):
    for p in range(npan):
        out_ref[p] = h[:, p * WP:(p + 1) * WP]


def _tc_layer1(agg, x, w_rel, w_root, b, r):
    n, d = x.shape
    dh = w_rel.shape[1]
    npan = agg.shape[1]

    def body(agg_ref, x_ref, wr_ref, wo_ref, b_ref, out_ref):
        a = _cat_partials(agg_ref)
        h = _dot(a, wr_ref[...]) + _dot(x_ref[...], wo_ref[...]) + b_ref[...]
        _split_store(out_ref, jnp.maximum(h, 0.0), dh // WP)

    return pl.pallas_call(
        body,
        grid=(n // r,),
        in_specs=[
            pl.BlockSpec((2, npan, r, WP), lambda i: (0, 0, i, 0)),
            pl.BlockSpec((r, d), lambda i: (i, 0)),
            pl.BlockSpec((d, dh), lambda i: (0, 0)),
            pl.BlockSpec((d, dh), lambda i: (0, 0)),
            pl.BlockSpec((1, dh), lambda i: (0, 0)),
        ],
        out_specs=pl.BlockSpec((dh // WP, r, WP), lambda i: (0, i, 0)),
        out_shape=jax.ShapeDtypeStruct((dh // WP, n, WP), jnp.float32),
    )(agg, x, w_rel, w_root, b[None])


def _tc_layer2(agg, h1s, w_rel, w_root, b, w3_rel_pad, r):
    hp, n, _ = h1s.shape
    d = hp * WP
    de = w_rel.shape[1]
    wc = w3_rel_pad.shape[1]
    npan = agg.shape[1]

    def body(agg_ref, h1_ref, wr_ref, wo_ref, b_ref, w3_ref, h2_ref, y3_ref):
        a = _cat_partials(agg_ref)
        h1 = jnp.concatenate([h1_ref[p] for p in range(hp)], axis=1)
        h = _dot(a, wr_ref[...]) + _dot(h1, wo_ref[...]) + b_ref[...]
        h2 = jnp.maximum(h, 0.0)
        h2_ref[...] = h2
        _split_store(y3_ref, _dot(h2, w3_ref[...]), wc // WP)

    return pl.pallas_call(
        body,
        grid=(n // r,),
        in_specs=[
            pl.BlockSpec((2, npan, r, WP), lambda i: (0, 0, i, 0)),
            pl.BlockSpec((hp, r, WP), lambda i: (0, i, 0)),
            pl.BlockSpec((d, de), lambda i: (0, 0)),
            pl.BlockSpec((d, de), lambda i: (0, 0)),
            pl.BlockSpec((1, de), lambda i: (0, 0)),
            pl.BlockSpec((de, wc), lambda i: (0, 0)),
        ],
        out_specs=[
            pl.BlockSpec((r, de), lambda i: (i, 0)),
            pl.BlockSpec((wc // WP, r, WP), lambda i: (0, i, 0)),
        ],
        out_shape=[
            jax.ShapeDtypeStruct((n, de), jnp.float32),
            jax.ShapeDtypeStruct((wc // WP, n, WP), jnp.float32),
        ],
    )(agg, h1s, w_rel, w_root, b[None], w3_rel_pad)


def _tc_layer3(agg, h2, w_root_pad, b_pad, n_classes, r):
    n, de = h2.shape
    wc = w_root_pad.shape[1]
    npan = agg.shape[1]

    def body(agg_ref, h2_ref, wo_ref, b_ref, out_ref):
        a = _cat_partials(agg_ref)
        logits = a + _dot(h2_ref[...], wo_ref[...]) + b_ref[...]
        mask = lax.broadcasted_iota(jnp.int32, logits.shape, 1) < n_classes
        masked = jnp.where(mask, logits, -jnp.inf)
        m = jnp.max(masked, axis=1, keepdims=True)
        e = jnp.where(mask, jnp.exp(logits - m), 0.0)
        s = jnp.sum(e, axis=1, keepdims=True)
        out_ref[...] = logits - m - jnp.log(s)

    return pl.pallas_call(
        body,
        grid=(n // r,),
        in_specs=[
            pl.BlockSpec((2, npan, r, WP), lambda i: (0, 0, i, 0)),
            pl.BlockSpec((r, de), lambda i: (i, 0)),
            pl.BlockSpec((de, wc), lambda i: (0, 0)),
            pl.BlockSpec((1, wc), lambda i: (0, 0)),
        ],
        out_specs=pl.BlockSpec((r, wc), lambda i: (i, 0)),
        out_shape=jax.ShapeDtypeStruct((n, wc), jnp.float32),
    )(agg, h2, w_root_pad, b_pad[None])


def kernel(x, edge_index, W1_rel, b1, W1_root, W2_rel, b2, W2_root, W3_rel, b3, W3_root):
    n, d_in = x.shape
    e = edge_index.shape[1]
    n_classes = W3_rel.shape[1]
    d_hid = W1_rel.shape[1]
    assert d_hid == d_in and d_in % WP == 0
    r = 1000
    assert n % r == 0

    epw = -(-e // (NW * NB * K)) * NB * K
    e_pad = epw * NW
    n_chunks = epw // K
    src = jnp.pad(edge_index[0], (0, e_pad - e)).reshape(NW, n_chunks, K)
    dst = jnp.pad(edge_index[1], (0, e_pad - e),
                  constant_values=n).reshape(NW, n_chunks, K)

    wc = -(-n_classes // (2 * WP)) * 2 * WP
    W3r = jnp.pad(W3_rel, ((0, 0), (0, wc - n_classes)))
    W3o = jnp.pad(W3_root, ((0, 0), (0, wc - n_classes)))
    b3p = jnp.pad(b3, (0, wc - n_classes))

    agg_wide = _make_agg(n, n_chunks, d_in // WP)
    agg_narrow = _make_agg(n, n_chunks, wc // WP)

    x4 = jnp.moveaxis(x.reshape(n, d_in // WP, WP), 1, 0)
    agg1 = agg_wide(x4, src, dst)
    h1s = _tc_layer1(agg1, x, W1_rel, W1_root, b1, r)
    agg2 = agg_wide(h1s, src, dst)
    h2, y3s = _tc_layer2(agg2, h1s, W2_rel, W2_root, b2, W3r, r)
    agg3 = agg_narrow(y3s, src, dst)
    out = _tc_layer3(agg3, h2, W3o, b3p, n_classes, r)
    return out[:, :n_classes]


# speedup vs baseline: 10.7590x; 1.2265x over previous
import functools

import jax
import jax.numpy as jnp
from jax import lax
from jax.experimental import pallas as pl
from jax.experimental.pallas import tpu as pltpu
from jax.experimental.pallas import tpu_sc as plsc

NC = 2
NS = 16
NW = NC * NS
L = 16
K = 128
WP = 32
YW = 128
NB = 4

_HIGHEST = lax.Precision.HIGHEST


@functools.lru_cache(maxsize=None)
def _make_agg(n_nodes, n_chunks, n_panels):
    assert n_chunks % NB == 0 and n_nodes % NS == 0
    stripe = -(-(n_nodes + 1) // (NS * K)) * K
    n_acc = stripe * NS
    rows_pt = n_nodes // NS

    mesh = plsc.VectorSubcoreMesh(core_axis_name="c", subcore_axis_name="s")

    def body(y_hbm, src_hbm, dst_hbm, out_hbm,
             srcv, dstv, rows, zbuf, acc, ysp, gsems, ssems):
        cid = lax.axis_index("c")
        sid = lax.axis_index("s")
        wid = sid * NC + cid

        @pl.loop(0, K)
        def _zero(r):
            for c in range(WP // L):
                zbuf[r, pl.ds(c * L, L)] = jnp.zeros((L,), jnp.float32)

        pltpu.sync_copy(src_hbm.at[wid], srcv)
        pltpu.sync_copy(dst_hbm.at[wid], dstv)

        def pipeline():
            def fire_g(j, b):
                pltpu.async_copy(ysp.at[srcv.at[j]], rows.at[b], gsems.at[b])

            def drain_g(b):
                pltpu.make_async_copy(ysp.at[srcv.at[0]], rows.at[b],
                                      gsems.at[b]).wait()

            def fire_s(j, b):
                pltpu.async_copy(rows.at[b], acc.at[dstv.at[j]], ssems.at[b],
                                 add=True)

            def drain_s(b):
                pltpu.make_async_copy(rows.at[b], acc.at[dstv.at[0]],
                                      ssems.at[b]).wait()

            for b in range(NB):
                fire_g(b, b)

            @pl.loop(0, n_chunks // NB - 1)
            def _main(q):
                t = NB * q
                for b in range(NB):
                    drain_g(b)
                    fire_s(t + b, b)
                    drain_s(b)
                    fire_g(t + b + NB, b)

            for b in range(NB):
                drain_g(b)
                fire_s(n_chunks - NB + b, b)
            for b in range(NB):
                drain_s(b)

        @pl.loop(0, n_panels)
        def _panel(p):
            for j in range(stripe // K):
                pltpu.sync_copy(zbuf, acc.at[pl.ds(sid * stripe + j * K, K)])

            r0 = sid * rows_pt
            pltpu.sync_copy(y_hbm.at[pl.ds(r0, rows_pt), pl.ds(p * WP, WP)],
                            ysp.at[pl.ds(r0, rows_pt)])

            plsc.subcore_barrier()
            pipeline()
            plsc.subcore_barrier()

            a0 = sid * stripe
            pltpu.sync_copy(acc.at[pl.ds(a0, stripe)],
                            out_hbm.at[cid, pl.ds(a0, stripe),
                                       pl.ds(p * WP, WP)])
            plsc.subcore_barrier()

    return pl.kernel(
        body,
        jax.ShapeDtypeStruct((NC, n_acc, YW), jnp.float32),
        mesh=mesh,
        compiler_params=pltpu.CompilerParams(use_tc_tiling_on_sc=False),
        scratch_types=[
            pltpu.VMEM((n_chunks, K), jnp.int32),
            pltpu.VMEM((n_chunks, K), jnp.int32),
            pltpu.VMEM((NB, K, WP), jnp.float32),
            pltpu.VMEM((K, WP), jnp.float32),
            pltpu.VMEM_SHARED((n_acc, WP), jnp.float32),
            pltpu.VMEM_SHARED((n_nodes, WP), jnp.float32),
            pltpu.SemaphoreType.DMA((NB,)),
            pltpu.SemaphoreType.DMA((NB,)),
        ],
    )


def _dot(a, b):
    return jnp.dot(a, b, preferred_element_type=jnp.float32, precision=_HIGHEST)


def _tc_layer1(agg, x, w_rel, w_root, b, r):
    n, d = x.shape
    dh = w_rel.shape[1]

    def body(agg_ref, x_ref, wr_ref, wo_ref, b_ref, out_ref):
        a = agg_ref[0] + agg_ref[1]
        h = _dot(a, wr_ref[...]) + _dot(x_ref[...], wo_ref[...]) + b_ref[...]
        out_ref[...] = jnp.maximum(h, 0.0)

    return pl.pallas_call(
        body,
        grid=(n // r,),
        in_specs=[
            pl.BlockSpec((2, r, d), lambda i: (0, i, 0)),
            pl.BlockSpec((r, d), lambda i: (i, 0)),
            pl.BlockSpec((d, dh), lambda i: (0, 0)),
            pl.BlockSpec((d, dh), lambda i: (0, 0)),
            pl.BlockSpec((1, dh), lambda i: (0, 0)),
        ],
        out_specs=pl.BlockSpec((r, dh), lambda i: (i, 0)),
        out_shape=jax.ShapeDtypeStruct((n, dh), jnp.float32),
    )(agg, x, w_rel, w_root, b[None])


def _tc_layer2(agg, h1, w_rel, w_root, b, w3_rel_pad, r):
    n, d = h1.shape
    de = w_rel.shape[1]
    wc = w3_rel_pad.shape[1]

    def body(agg_ref, h1_ref, wr_ref, wo_ref, b_ref, w3_ref, h2_ref, y3_ref):
        a = agg_ref[0] + agg_ref[1]
        h = _dot(a, wr_ref[...]) + _dot(h1_ref[...], wo_ref[...]) + b_ref[...]
        h2 = jnp.maximum(h, 0.0)
        h2_ref[...] = h2
        y3_ref[...] = _dot(h2, w3_ref[...])

    return pl.pallas_call(
        body,
        grid=(n // r,),
        in_specs=[
            pl.BlockSpec((2, r, d), lambda i: (0, i, 0)),
            pl.BlockSpec((r, d), lambda i: (i, 0)),
            pl.BlockSpec((d, de), lambda i: (0, 0)),
            pl.BlockSpec((d, de), lambda i: (0, 0)),
            pl.BlockSpec((1, de), lambda i: (0, 0)),
            pl.BlockSpec((de, wc), lambda i: (0, 0)),
        ],
        out_specs=[
            pl.BlockSpec((r, de), lambda i: (i, 0)),
            pl.BlockSpec((r, wc), lambda i: (i, 0)),
        ],
        out_shape=[
            jax.ShapeDtypeStruct((n, de), jnp.float32),
            jax.ShapeDtypeStruct((n, wc), jnp.float32),
        ],
    )(agg, h1, w_rel, w_root, b[None], w3_rel_pad)


def _tc_layer3(agg, h2, w_root_pad, b_pad, n_classes, wc, r):
    n, de = h2.shape

    def body(agg_ref, h2_ref, wo_ref, b_ref, out_ref):
        a = (agg_ref[0] + agg_ref[1])[:, :wc]
        logits = a + _dot(h2_ref[...], wo_ref[...]) + b_ref[...]
        mask = lax.broadcasted_iota(jnp.int32, logits.shape, 1) < n_classes
        masked = jnp.where(mask, logits, -jnp.inf)
        m = jnp.max(masked, axis=1, keepdims=True)
        e = jnp.where(mask, jnp.exp(logits - m), 0.0)
        s = jnp.sum(e, axis=1, keepdims=True)
        out_ref[...] = logits - m - jnp.log(s)

    return pl.pallas_call(
        body,
        grid=(n // r,),
        in_specs=[
            pl.BlockSpec((2, r, YW), lambda i: (0, i, 0)),
            pl.BlockSpec((r, de), lambda i: (i, 0)),
            pl.BlockSpec((de, wc), lambda i: (0, 0)),
            pl.BlockSpec((1, wc), lambda i: (0, 0)),
        ],
        out_specs=pl.BlockSpec((r, wc), lambda i: (i, 0)),
        out_shape=jax.ShapeDtypeStruct((n, wc), jnp.float32),
    )(agg, h2, w_root_pad, b_pad[None])


def kernel(x, edge_index, W1_rel, b1, W1_root, W2_rel, b2, W2_root, W3_rel, b3, W3_root):
    n, d_in = x.shape
    e = edge_index.shape[1]
    n_classes = W3_rel.shape[1]
    d_hid = W1_rel.shape[1]
    assert d_hid == d_in == YW and d_in % WP == 0
    r = 1000
    assert n % r == 0

    epw = -(-e // (NW * NB * K)) * NB * K
    e_pad = epw * NW
    n_chunks = epw // K
    src = jnp.pad(edge_index[0], (0, e_pad - e)).reshape(NW, n_chunks, K)
    dst = jnp.pad(edge_index[1], (0, e_pad - e),
                  constant_values=n).reshape(NW, n_chunks, K)

    wc = -(-n_classes // (2 * WP)) * 2 * WP
    W3r = jnp.pad(W3_rel, ((0, 0), (0, YW - n_classes)))
    W3o = jnp.pad(W3_root, ((0, 0), (0, wc - n_classes)))
    b3p = jnp.pad(b3, (0, wc - n_classes))

    agg_wide = _make_agg(n, n_chunks, d_in // WP)
    agg_narrow = _make_agg(n, n_chunks, wc // WP)

    agg1 = agg_wide(x, src, dst)
    h1 = _tc_layer1(agg1, x, W1_rel, W1_root, b1, r)
    agg2 = agg_wide(h1, src, dst)
    h2, y3 = _tc_layer2(agg2, h1, W2_rel, W2_root, b2, W3r, r)
    agg3 = agg_narrow(y3, src, dst)
    out = _tc_layer3(agg3, h2, W3o, b3p, n_classes, wc, r)
    return out[:, :n_classes]


# speedup vs baseline: 11.6015x; 1.0783x over previous
import functools

import jax
import jax.numpy as jnp
from jax import lax
from jax.experimental import pallas as pl
from jax.experimental.pallas import tpu as pltpu
from jax.experimental.pallas import tpu_sc as plsc

NC = 2
NS = 16
NW = NC * NS
L = 16
K = 128
WP = 32
YW = 128
NB = 4

_HIGHEST = lax.Precision.HIGHEST


@functools.lru_cache(maxsize=None)
def _make_agg(n_nodes, n_chunks, n_panels):
    assert n_chunks % NB == 0 and n_nodes % NS == 0
    stripe = -(-(n_nodes + 1) // (NS * K)) * K
    n_acc = stripe * NS
    rows_pt = n_nodes // NS

    mesh = plsc.VectorSubcoreMesh(core_axis_name="c", subcore_axis_name="s")

    def body(y_hbm, src_hbm, dst_hbm, out_hbm,
             srcv, dstv, rows, zbuf, acc, ysp, gsems, ssems):
        cid = lax.axis_index("c")
        sid = lax.axis_index("s")
        wid = sid * NC + cid

        @pl.loop(0, K)
        def _zero(r):
            for c in range(WP // L):
                zbuf[r, pl.ds(c * L, L)] = jnp.zeros((L,), jnp.float32)

        pltpu.sync_copy(src_hbm.at[wid], srcv)
        pltpu.sync_copy(dst_hbm.at[wid], dstv)

        def pipeline():
            def fire_g(j, b):
                pltpu.async_copy(ysp.at[srcv.at[j]], rows.at[b], gsems.at[b])

            def drain_g(b):
                pltpu.make_async_copy(ysp.at[srcv.at[0]], rows.at[b],
                                      gsems.at[b]).wait()

            def fire_s(j, b):
                pltpu.async_copy(rows.at[b], acc.at[dstv.at[j]], ssems.at[b],
                                 add=True)

            def drain_s(b):
                pltpu.make_async_copy(rows.at[b], acc.at[dstv.at[0]],
                                      ssems.at[b]).wait()

            for b in range(NB):
                fire_g(b, b)

            @pl.loop(0, n_chunks // NB - 1)
            def _main(q):
                t = NB * q
                for b in range(NB):
                    drain_g(b)
                    fire_s(t + b, b)
                    drain_s(b)
                    fire_g(t + b + NB, b)

            for b in range(NB):
                drain_g(b)
                fire_s(n_chunks - NB + b, b)
            for b in range(NB):
                drain_s(b)

        @pl.loop(0, n_panels)
        def _panel(p):
            for j in range(stripe // K):
                pltpu.sync_copy(zbuf, acc.at[pl.ds(sid * stripe + j * K, K)])

            r0 = sid * rows_pt
            pltpu.sync_copy(y_hbm.at[pl.ds(r0, rows_pt), pl.ds(p * WP, WP)],
                            ysp.at[pl.ds(r0, rows_pt)])

            plsc.subcore_barrier()
            pipeline()
            plsc.subcore_barrier()

            a0 = sid * stripe
            pltpu.sync_copy(acc.at[pl.ds(a0, stripe)],
                            out_hbm.at[cid, pl.ds(a0, stripe),
                                       pl.ds(p * WP, WP)])
            plsc.subcore_barrier()

    return pl.kernel(
        body,
        jax.ShapeDtypeStruct((NC, n_acc, YW), jnp.float32),
        mesh=mesh,
        compiler_params=pltpu.CompilerParams(use_tc_tiling_on_sc=False),
        scratch_types=[
            pltpu.VMEM((n_chunks, K), jnp.int32),
            pltpu.VMEM((n_chunks, K), jnp.int32),
            pltpu.VMEM((NB, K, WP), jnp.float32),
            pltpu.VMEM((K, WP), jnp.float32),
            pltpu.VMEM_SHARED((n_acc, WP), jnp.float32),
            pltpu.VMEM_SHARED((n_nodes, WP), jnp.float32),
            pltpu.SemaphoreType.DMA((NB,)),
            pltpu.SemaphoreType.DMA((NB,)),
        ],
    )


def _dot(a, b):
    return jnp.dot(a, b, preferred_element_type=jnp.float32)


def _tc_layer1(agg, x, w_rel, w_root, b, r):
    n, d = x.shape
    dh = w_rel.shape[1]

    def body(agg_ref, x_ref, wr_ref, wo_ref, b_ref, out_ref):
        a = agg_ref[0] + agg_ref[1]
        h = _dot(a, wr_ref[...]) + _dot(x_ref[...], wo_ref[...]) + b_ref[...]
        out_ref[...] = jnp.maximum(h, 0.0)

    return pl.pallas_call(
        body,
        grid=(n // r,),
        in_specs=[
            pl.BlockSpec((2, r, d), lambda i: (0, i, 0)),
            pl.BlockSpec((r, d), lambda i: (i, 0)),
            pl.BlockSpec((d, dh), lambda i: (0, 0)),
            pl.BlockSpec((d, dh), lambda i: (0, 0)),
            pl.BlockSpec((1, dh), lambda i: (0, 0)),
        ],
        out_specs=pl.BlockSpec((r, dh), lambda i: (i, 0)),
        out_shape=jax.ShapeDtypeStruct((n, dh), jnp.float32),
    )(agg, x, w_rel, w_root, b[None])


def _tc_layer2(agg, h1, w_rel, w_root, b, w3_rel_pad, r):
    n, d = h1.shape
    de = w_rel.shape[1]
    wc = w3_rel_pad.shape[1]

    def body(agg_ref, h1_ref, wr_ref, wo_ref, b_ref, w3_ref, h2_ref, y3_ref):
        a = agg_ref[0] + agg_ref[1]
        h = _dot(a, wr_ref[...]) + _dot(h1_ref[...], wo_ref[...]) + b_ref[...]
        h2 = jnp.maximum(h, 0.0)
        h2_ref[...] = h2
        y3_ref[...] = _dot(h2, w3_ref[...])

    return pl.pallas_call(
        body,
        grid=(n // r,),
        in_specs=[
            pl.BlockSpec((2, r, d), lambda i: (0, i, 0)),
            pl.BlockSpec((r, d), lambda i: (i, 0)),
            pl.BlockSpec((d, de), lambda i: (0, 0)),
            pl.BlockSpec((d, de), lambda i: (0, 0)),
            pl.BlockSpec((1, de), lambda i: (0, 0)),
            pl.BlockSpec((de, wc), lambda i: (0, 0)),
        ],
        out_specs=[
            pl.BlockSpec((r, de), lambda i: (i, 0)),
            pl.BlockSpec((r, wc), lambda i: (i, 0)),
        ],
        out_shape=[
            jax.ShapeDtypeStruct((n, de), jnp.float32),
            jax.ShapeDtypeStruct((n, wc), jnp.float32),
        ],
    )(agg, h1, w_rel, w_root, b[None], w3_rel_pad)


def _tc_layer3(agg, h2, w_root_pad, b_pad, n_classes, wc, r):
    n, de = h2.shape

    def body(agg_ref, h2_ref, wo_ref, b_ref, out_ref):
        a = (agg_ref[0] + agg_ref[1])[:, :wc]
        logits = a + _dot(h2_ref[...], wo_ref[...]) + b_ref[...]
        mask = lax.broadcasted_iota(jnp.int32, logits.shape, 1) < n_classes
        masked = jnp.where(mask, logits, -jnp.inf)
        m = jnp.max(masked, axis=1, keepdims=True)
        e = jnp.where(mask, jnp.exp(logits - m), 0.0)
        s = jnp.sum(e, axis=1, keepdims=True)
        out_ref[...] = logits - m - jnp.log(s)

    return pl.pallas_call(
        body,
        grid=(n // r,),
        in_specs=[
            pl.BlockSpec((2, r, YW), lambda i: (0, i, 0)),
            pl.BlockSpec((r, de), lambda i: (i, 0)),
            pl.BlockSpec((de, wc), lambda i: (0, 0)),
            pl.BlockSpec((1, wc), lambda i: (0, 0)),
        ],
        out_specs=pl.BlockSpec((r, wc), lambda i: (i, 0)),
        out_shape=jax.ShapeDtypeStruct((n, wc), jnp.float32),
    )(agg, h2, w_root_pad, b_pad[None])


def kernel(x, edge_index, W1_rel, b1, W1_root, W2_rel, b2, W2_root, W3_rel, b3, W3_root):
    n, d_in = x.shape
    e = edge_index.shape[1]
    n_classes = W3_rel.shape[1]
    d_hid = W1_rel.shape[1]
    assert d_hid == d_in == YW and d_in % WP == 0
    r = 1000
    assert n % r == 0

    epw = -(-e // (NW * NB * K)) * NB * K
    e_pad = epw * NW
    n_chunks = epw // K
    src = jnp.pad(edge_index[0], (0, e_pad - e)).reshape(NW, n_chunks, K)
    dst = jnp.pad(edge_index[1], (0, e_pad - e),
                  constant_values=n).reshape(NW, n_chunks, K)

    wc = -(-n_classes // (2 * WP)) * 2 * WP
    W3r = jnp.pad(W3_rel, ((0, 0), (0, YW - n_classes)))
    W3o = jnp.pad(W3_root, ((0, 0), (0, wc - n_classes)))
    b3p = jnp.pad(b3, (0, wc - n_classes))

    agg_wide = _make_agg(n, n_chunks, d_in // WP)
    agg_narrow = _make_agg(n, n_chunks, wc // WP)

    agg1 = agg_wide(x, src, dst)
    h1 = _tc_layer1(agg1, x, W1_rel, W1_root, b1, r)
    agg2 = agg_wide(h1, src, dst)
    h2, y3 = _tc_layer2(agg2, h1, W2_rel, W2_root, b2, W3r, r)
    agg3 = agg_narrow(y3, src, dst)
    out = _tc_layer3(agg3, h2, W3o, b3p, n_classes, wc, r)
    return out[:, :n_classes]
